# baseline (speedup 1.0000x reference)
import functools

import jax
import jax.numpy as jnp
from jax import lax
from jax.experimental import pallas as pl
from jax.experimental.pallas import tpu as pltpu
from jax.experimental.pallas import tpu_sc as plsc

_NC = 2
_NS = 16
_NW = _NC * _NS
_CH = 128


def _sc_word_gather(widx, table):
    n = widx.shape[0]
    per = n // _NW
    nch = per // _CH
    mesh = plsc.VectorSubcoreMesh(core_axis_name="c", subcore_axis_name="s")

    @functools.partial(
        pl.kernel,
        mesh=mesh,
        out_type=jax.ShapeDtypeStruct((n, 256), jnp.float32),
        scratch_types=[
            pltpu.VMEM((per,), jnp.int32),
            pltpu.VMEM((2, _CH, 128), jnp.float32),
            pltpu.SemaphoreType.DMA,
            pltpu.SemaphoreType.DMA,
        ],
    )
    def k(idx_hbm, tab_hbm, out_hbm, idx_v, rows_v, g0, g1):
        wid = lax.axis_index("s") * _NC + lax.axis_index("c")
        base = wid * per
        pltpu.sync_copy(idx_hbm.at[pl.ds(base, per)], idx_v)
        gs = (g0, g1)

        def gather(j, slot):
            pltpu.async_copy(
                tab_hbm.at[idx_v.at[pl.ds(j * _CH, _CH)]], rows_v.at[slot], gs[slot]
            )

        def gwait(j, slot):
            pltpu.make_async_copy(
                tab_hbm.at[idx_v.at[pl.ds(j * _CH, _CH)]], rows_v.at[slot], gs[slot]
            ).wait()

        def store(j, slot):
            pltpu.sync_copy(
                rows_v.at[slot],
                out_hbm.at[pl.ds(base + j * _CH, _CH), pl.ds(0, 128)],
            )

        gather(0, 0)
        gather(1, 1)

        def body(p, carry):
            j = 2 * p
            gwait(j, 0)
            store(j, 0)

            @pl.when(j + 2 < nch)
            def _():
                gather(j + 2, 0)

            gwait(j + 1, 1)
            store(j + 1, 1)

            @pl.when(j + 3 < nch)
            def _():
                gather(j + 3, 1)

            return carry

        lax.fori_loop(0, nch // 2, body, 0)

    return k(widx, table)


def _tdnn_body(cit_ref, wide_ref, ctt_ref, w1_ref, w2_ref, w3_ref, out_ref):
    del wide_ref
    r = cit_ref.shape[1]
    cit = cit_ref[...]
    subs = lax.broadcasted_iota(jnp.int32, (128, r), 0)
    ctt = ctt_ref[...]

    es = []
    for t in range(12):
        idx_t = jnp.broadcast_to(cit[t : t + 1, :], (16, r))
        es.append(jnp.take_along_axis(ctt, idx_t, axis=1))
    eT = jnp.concatenate(es, axis=1)

    y1 = jnp.dot(w1_ref[...], eT, preferred_element_type=jnp.float32)
    m1 = y1[:, 0:r]
    for t in range(1, 12):
        m1 = jnp.maximum(m1, y1[:, t * r : (t + 1) * r])
    m1 = jnp.maximum(m1, 0.0)

    x2 = jnp.concatenate([eT[:, 0 : 11 * r], eT[:, r : 12 * r]], axis=0)
    y2 = jnp.dot(w2_ref[...], x2, preferred_element_type=jnp.float32)
    m2 = y2[:, 0:r]
    for t in range(1, 11):
        m2 = jnp.maximum(m2, y2[:, t * r : (t + 1) * r])
    m2 = jnp.maximum(m2, 0.0)

    x3 = jnp.concatenate(
        [eT[:, 0 : 10 * r], eT[:, r : 11 * r], eT[:, 2 * r : 12 * r]], axis=0
    )
    y3 = jnp.dot(w3_ref[...], x3, preferred_element_type=jnp.float32)
    m3 = y3[:, 0:r]
    for t in range(1, 10):
        m3 = jnp.maximum(m3, y3[:, t * r : (t + 1) * r])
    m3 = jnp.maximum(m3, 0.0)

    s = jnp.concatenate([m1, m2, m3], axis=0)
    out_ref[...] = s.T


def _tdnn_call(cit, wide, ctt, w1, w2, w3):
    n = cit.shape[1]
    r = 2048
    return pl.pallas_call(
        _tdnn_body,
        grid=(n // r,),
        in_specs=[
            pl.BlockSpec((12, r), lambda i: (0, i)),
            pl.BlockSpec(memory_space=pl.ANY),
            pl.BlockSpec((16, 128), lambda i: (0, 0)),
            pl.BlockSpec((32, 16), lambda i: (0, 0)),
            pl.BlockSpec((32, 32), lambda i: (0, 0)),
            pl.BlockSpec((64, 48), lambda i: (0, 0)),
        ],
        out_specs=pl.BlockSpec((r, 128), lambda i: (i, 1)),
        out_shape=jax.ShapeDtypeStruct((n, 256), jnp.float32),
        input_output_aliases={1: 0},
    )(cit, wide, ctt, w1, w2, w3)


def kernel(word_input, character_input, word_table, char_table, W1, W2, W3):
    b, s = word_input.shape
    n = b * s
    widx = word_input.reshape(n).astype(jnp.int32)
    wide = _sc_word_gather(widx, word_table)

    cit = character_input.reshape(n, 12).astype(jnp.int32).T
    ctt = char_table.T
    w2 = jnp.concatenate([W2[:, :, 0], W2[:, :, 1]], axis=1)
    w3 = jnp.concatenate([W3[:, :, 0], W3[:, :, 1], W3[:, :, 2]], axis=1)
    out = _tdnn_call(cit, wide, ctt, W1[:, :, 0], w2, w3)
    return out.reshape(b, s, 256)


# speedup vs baseline: 1.0152x; 1.0152x over previous
import functools

import jax
import jax.numpy as jnp
from jax import lax
from jax.experimental import pallas as pl
from jax.experimental.pallas import tpu as pltpu
from jax.experimental.pallas import tpu_sc as plsc

_NC = 2
_NS = 16
_NW = _NC * _NS
_CH = 128


def _sc_word_gather(widx, table):
    n = widx.shape[0]
    per = n // _NW
    nch = per // _CH
    mesh = plsc.VectorSubcoreMesh(core_axis_name="c", subcore_axis_name="s")

    @functools.partial(
        pl.kernel,
        mesh=mesh,
        out_type=jax.ShapeDtypeStruct((n, 256), jnp.float32),
        scratch_types=[
            pltpu.VMEM((per,), jnp.int32),
            pltpu.VMEM((2, _CH, 128), jnp.float32),
            pltpu.SemaphoreType.DMA,
            pltpu.SemaphoreType.DMA,
        ],
    )
    def k(idx_hbm, tab_hbm, out_hbm, idx_v, rows_v, g0, g1):
        wid = lax.axis_index("s") * _NC + lax.axis_index("c")
        base = wid * per
        pltpu.sync_copy(idx_hbm.at[pl.ds(base, per)], idx_v)
        gs = (g0, g1)

        def gather(j, slot):
            pltpu.async_copy(
                tab_hbm.at[idx_v.at[pl.ds(j * _CH, _CH)]], rows_v.at[slot], gs[slot]
            )

        def gwait(j, slot):
            pltpu.make_async_copy(
                tab_hbm.at[idx_v.at[pl.ds(j * _CH, _CH)]], rows_v.at[slot], gs[slot]
            ).wait()

        def store(j, slot):
            pltpu.sync_copy(
                rows_v.at[slot],
                out_hbm.at[pl.ds(base + j * _CH, _CH), pl.ds(0, 128)],
            )

        gather(0, 0)
        gather(1, 1)

        def body(p, carry):
            j = 2 * p
            gwait(j, 0)
            store(j, 0)

            @pl.when(j + 2 < nch)
            def _():
                gather(j + 2, 0)

            gwait(j + 1, 1)
            store(j + 1, 1)

            @pl.when(j + 3 < nch)
            def _():
                gather(j + 3, 1)

            return carry

        lax.fori_loop(0, nch // 2, body, 0)

    return k(widx, table)


def _tdnn_body(cit_ref, wide_ref, ctt_ref, w1_ref, w2_ref, w3_ref, out_ref):
    del wide_ref
    r = cit_ref.shape[1]
    cit = cit_ref[...]
    subs = lax.broadcasted_iota(jnp.int32, (128, r), 0)
    ctt = ctt_ref[...]

    es = []
    for t in range(12):
        idx_t = jnp.broadcast_to(cit[t : t + 1, :], (16, r))
        es.append(jnp.take_along_axis(ctt, idx_t, axis=1))
    eT = jnp.concatenate(es, axis=1)

    y1 = jnp.maximum(
        jnp.dot(w1_ref[...], eT, preferred_element_type=jnp.float32), 0.0
    )
    m1 = y1[:, 0:r]
    for t in range(1, 12):
        m1 = jnp.maximum(m1, y1[:, t * r : (t + 1) * r])

    x2 = jnp.concatenate([eT[:, 0 : 11 * r], eT[:, r : 12 * r]], axis=0)
    y2 = jnp.maximum(
        jnp.dot(w2_ref[...], x2, preferred_element_type=jnp.float32), 0.0
    )
    m2 = y2[:, 0:r]
    for t in range(1, 11):
        m2 = jnp.maximum(m2, y2[:, t * r : (t + 1) * r])

    x3 = jnp.concatenate(
        [eT[:, 0 : 10 * r], eT[:, r : 11 * r], eT[:, 2 * r : 12 * r]], axis=0
    )
    y3 = jnp.maximum(
        jnp.dot(w3_ref[...], x3, preferred_element_type=jnp.float32), 0.0
    )
    m3 = y3[:, 0:r]
    for t in range(1, 10):
        m3 = jnp.maximum(m3, y3[:, t * r : (t + 1) * r])

    s = jnp.concatenate([m1, m2, m3], axis=0)
    out_ref[...] = s.T


def _tdnn_call(cit, wide, ctt, w1, w2, w3):
    n = cit.shape[1]
    r = 2048
    return pl.pallas_call(
        _tdnn_body,
        grid=(n // r,),
        in_specs=[
            pl.BlockSpec((12, r), lambda i: (0, i)),
            pl.BlockSpec(memory_space=pl.ANY),
            pl.BlockSpec((16, 128), lambda i: (0, 0)),
            pl.BlockSpec((32, 16), lambda i: (0, 0)),
            pl.BlockSpec((32, 32), lambda i: (0, 0)),
            pl.BlockSpec((64, 48), lambda i: (0, 0)),
        ],
        out_specs=pl.BlockSpec((r, 128), lambda i: (i, 1)),
        out_shape=jax.ShapeDtypeStruct((n, 256), jnp.float32),
        input_output_aliases={1: 0},
    )(cit, wide, ctt, w1, w2, w3)


def kernel(word_input, character_input, word_table, char_table, W1, W2, W3):
    b, s = word_input.shape
    n = b * s
    widx = word_input.reshape(n).astype(jnp.int32)
    wide = _sc_word_gather(widx, word_table)

    cit = character_input.reshape(n, 12).astype(jnp.int32).T
    ctt = char_table.T
    w2 = jnp.concatenate([W2[:, :, 0], W2[:, :, 1]], axis=1)
    w3 = jnp.concatenate([W3[:, :, 0], W3[:, :, 1], W3[:, :, 2]], axis=1)
    out = _tdnn_call(cit, wide, ctt, W1[:, :, 0], w2, w3)
    return out.reshape(b, s, 256)
